# baseline (speedup 1.0000x reference)
import functools

import jax
import jax.numpy as jnp
from jax import lax
from jax.experimental import pallas as pl
from jax.experimental.pallas import tpu as pltpu
from jax.experimental.pallas import tpu_sc as plsc

N = 100000
E = 1600000
D = 128
H = 64
C = 32
K = 10
ALPHA = 0.1

NCORE = 2
NTILE = 16
NPT = 6256
N_PAD = NTILE * NPT
ROWS_PT = 792
EROWS = NTILE * ROWS_PT
E_PAD = EROWS * 128
SW = 2
NBUF = 4
NWIN = ROWS_PT // SW
NTRI = NWIN // NBUF
DSW = 8
NSW = ROWS_PT // DSW
UCH = 92
NUCH = NPT // UCH
NUPAIR = NUCH // 2

MLP_BLK = 4352
OUT_BLK = 2000

_MESH = dict(core_axis_name="c", subcore_axis_name="s", num_cores=NCORE,
             num_subcores=NTILE)


def _mlp_body(x_ref, w1_ref, b1_ref, w2_ref, b2_ref, out_ref):
    h = jnp.maximum(
        jnp.dot(x_ref[...], w1_ref[...], preferred_element_type=jnp.float32)
        + b1_ref[...], 0.0)
    out_ref[...] = (
        jnp.dot(h, w2_ref[...], preferred_element_type=jnp.float32)
        + b2_ref[...])


def _mlp(xp, W1, b1, W2, b2):
    return pl.pallas_call(
        _mlp_body,
        grid=(N_PAD // MLP_BLK,),
        in_specs=[
            pl.BlockSpec((MLP_BLK, D), lambda i: (i, 0)),
            pl.BlockSpec((D, H), lambda i: (0, 0)),
            pl.BlockSpec((1, H), lambda i: (0, 0)),
            pl.BlockSpec((H, C), lambda i: (0, 0)),
            pl.BlockSpec((1, C), lambda i: (0, 0)),
        ],
        out_specs=pl.BlockSpec((MLP_BLK, C), lambda i: (i, 0)),
        out_shape=jax.ShapeDtypeStruct((N_PAD, C), jnp.float32),
    )(xp, W1, b1[None, :], W2, b2[None, :])


def _deg_body(dst2, zeros_n, deg_out, dacc, didx, ones_v, zbuf):
    cid = lax.axis_index("c")
    sid = lax.axis_index("s")

    @pl.when(cid == 0)
    def _():
        node_lo = sid * NPT
        pltpu.sync_copy(zeros_n.at[pl.ds(node_lo, NPT)], zbuf)
        pltpu.sync_copy(zbuf, dacc.at[pl.ds(node_lo, NPT)])
        for u in range(8):
            ones_v[pl.ds(u * 16, 16)] = jnp.full((16,), 1.0, jnp.float32)
        plsc.subcore_barrier()
        row0 = sid * ROWS_PT

        def swin(i, carry):
            r = row0 + i * DSW
            pltpu.sync_copy(dst2.at[pl.ds(r, DSW)], didx)
            for j in range(DSW):
                pltpu.sync_copy(ones_v, dacc.at[didx.at[j]], add=True)
            return carry

        lax.fori_loop(0, NSW, swin, 0)
        plsc.subcore_barrier()
        pltpu.sync_copy(dacc.at[pl.ds(node_lo, NPT)], zbuf)
        pltpu.sync_copy(zbuf, deg_out.at[pl.ds(node_lo, NPT)])


@functools.cache
def _deg():
    return pl.kernel(
        _deg_body,
        out_type=jax.ShapeDtypeStruct((N_PAD,), jnp.float32),
        mesh=plsc.VectorSubcoreMesh(**_MESH),
        compiler_params=pltpu.CompilerParams(use_tc_tiling_on_sc=False),
        scratch_types=[
            pltpu.VMEM_SHARED((N_PAD,), jnp.float32),
            pltpu.VMEM((DSW, 128), jnp.int32),
            pltpu.VMEM((128,), jnp.float32),
            pltpu.VMEM((NPT,), jnp.float32),
        ],
    )


def _prep_body(h_ref, dg_ref, w0_ref, w1_ref, g0_ref, g1_ref, sb_ref,
               dsq_ref):
    deg = dg_ref[...] + 1.0
    dinv = jax.lax.rsqrt(deg)
    w = dinv * h_ref[...]
    w0_ref[...] = w[:, :16]
    w1_ref[...] = w[:, 16:]
    g0_ref[...] = ALPHA * w[:, :16]
    g1_ref[...] = ALPHA * w[:, 16:]
    sb_ref[...] = jnp.broadcast_to((1.0 - ALPHA) * dinv * dinv,
                                   (MLP_BLK, 16))
    dsq_ref[...] = deg * dinv


def _prep(h, dg):
    f32 = jnp.float32
    return pl.pallas_call(
        _prep_body,
        grid=(N_PAD // MLP_BLK,),
        in_specs=[
            pl.BlockSpec((MLP_BLK, C), lambda i: (i, 0)),
            pl.BlockSpec((MLP_BLK, 1), lambda i: (i, 0)),
        ],
        out_specs=[
            pl.BlockSpec((MLP_BLK, 16), lambda i: (i, 0)),
            pl.BlockSpec((MLP_BLK, 16), lambda i: (i, 0)),
            pl.BlockSpec((MLP_BLK, 16), lambda i: (i, 0)),
            pl.BlockSpec((MLP_BLK, 16), lambda i: (i, 0)),
            pl.BlockSpec((MLP_BLK, 16), lambda i: (i, 0)),
            pl.BlockSpec((MLP_BLK, 1), lambda i: (i, 0)),
        ],
        out_shape=[
            jax.ShapeDtypeStruct((N_PAD, 16), f32),
            jax.ShapeDtypeStruct((N_PAD, 16), f32),
            jax.ShapeDtypeStruct((N_PAD, 16), f32),
            jax.ShapeDtypeStruct((N_PAD, 16), f32),
            jax.ShapeDtypeStruct((N_PAD, 16), f32),
            jax.ShapeDtypeStruct((N_PAD, 1), f32),
        ],
    )(h, dg)


def _prop_body(src1, dst1, sb, gb, w_init, w_out,
               acc, sidx, didx, rows, abuf, sbuf, gbuf, *sems):
    cid = lax.axis_index("c")
    sid = lax.axis_index("s")
    node_lo = sid * NPT
    sisems = sems[:NBUF]
    disems = sems[NBUF:2 * NBUF]
    gsems = sems[2 * NBUF:3 * NBUF]
    ssems = sems[3 * NBUF:4 * NBUF]
    usems = sems[4 * NBUF:4 * NBUF + 2]
    osems = sems[4 * NBUF + 2:4 * NBUF + 4]

    def init(ci, c2):
        lo = node_lo + ci * UCH
        pltpu.sync_copy(w_init.at[cid].at[pl.ds(lo, UCH)], abuf.at[0])
        pltpu.sync_copy(abuf.at[0], acc.at[pl.ds(lo, UCH)])
        pltpu.sync_copy(abuf.at[0], w_out.at[cid].at[pl.ds(lo, UCH)])
        return c2

    lax.fori_loop(0, NUCH, init, 0)
    plsc.subcore_barrier()
    row0 = sid * ROWS_PT
    tbl = w_out.at[cid]

    WB = SW * 128

    def pre_sidx(w, b):
        e0 = (row0 + w * SW) * 128
        pltpu.async_copy(src1.at[pl.ds(e0, WB)], sidx.at[b], sisems[b])

    def wait_sidx(b):
        pltpu.make_async_copy(src1.at[pl.ds(0, WB)], sidx.at[b],
                              sisems[b]).wait()

    def drain_scatter(b):
        pltpu.make_async_copy(rows.at[b], acc.at[didx.at[b]],
                              ssems[b]).wait()

    def step(k, carry):
        for b in range(NBUF):
            pre_sidx(b, b)

        def tri(i, c2):
            gds = []
            dds = []
            for b in range(NBUF):
                wait_sidx(b)

                @pl.when(i > 0)
                def _(b=b):
                    drain_scatter(b)

                e0 = (row0 + (NBUF * i + b) * SW) * 128
                dds.append(pltpu.async_copy(dst1.at[pl.ds(e0, WB)],
                                            didx.at[b], disems[b]))
                gds.append(pltpu.async_copy(tbl.at[sidx.at[b]],
                                            rows.at[b], gsems[b]))
            for b in range(NBUF):
                gds[b].wait()
                dds[b].wait()
                pltpu.async_copy(rows.at[b], acc.at[didx.at[b]],
                                 ssems[b], add=True)

                @pl.when(i < NTRI - 1)
                def _(b=b):
                    pre_sidx(NBUF * i + b + NBUF, b)

            return c2

        lax.fori_loop(0, NTRI, tri, 0)
        for b in range(NBUF):
            drain_scatter(b)
        plsc.subcore_barrier()

        def issue_in(ci, b):
            lo = node_lo + ci * UCH
            return [
                pltpu.async_copy(sb.at[pl.ds(lo, UCH)], sbuf.at[b],
                                 usems[b]),
                pltpu.async_copy(gb.at[cid].at[pl.ds(lo, UCH)], gbuf.at[b],
                                 usems[b]),
            ]

        def compute(b):
            ab, sbf, gbf = abuf.at[b], sbuf.at[b], gbuf.at[b]

            def rowblk(jb, c3):
                for u in range(4):
                    i = jb * 4 + u
                    ab[i] = sbf[i] * ab[i] + gbf[i]
                return c3

            lax.fori_loop(0, UCH // 4, rowblk, 0)

        def issue_out(ci, b):
            lo = node_lo + ci * UCH
            d_ = pltpu.async_copy(abuf.at[b],
                                  w_out.at[cid].at[pl.ds(lo, UCH)],
                                  osems[b])
            pltpu.sync_copy(abuf.at[b], acc.at[pl.ds(lo, UCH)])
            return [d_]

        def upair(i, c2):
            din0 = issue_in(2 * i, 0)
            din1 = issue_in(2 * i + 1, 1)
            pltpu.sync_copy(acc.at[pl.ds(node_lo + 2 * i * UCH, UCH)],
                            abuf.at[0])
            pltpu.sync_copy(acc.at[pl.ds(node_lo + (2 * i + 1) * UCH, UCH)],
                            abuf.at[1])
            for d_ in din0:
                d_.wait()
            compute(0)
            dout0 = issue_out(2 * i, 0)
            for d_ in din1:
                d_.wait()
            compute(1)
            dout1 = issue_out(2 * i + 1, 1)
            for d_ in dout0:
                d_.wait()
            for d_ in dout1:
                d_.wait()
            return c2

        lax.fori_loop(0, NUPAIR, upair, 0)
        plsc.subcore_barrier()
        return carry

    lax.fori_loop(0, K, step, 0)


@functools.cache
def _prop():
    return pl.kernel(
        _prop_body,
        out_type=jax.ShapeDtypeStruct((NCORE, N_PAD, 16), jnp.float32),
        mesh=plsc.VectorSubcoreMesh(**_MESH),
        compiler_params=pltpu.CompilerParams(use_tc_tiling_on_sc=False),
        scratch_types=[
            pltpu.VMEM_SHARED((N_PAD, 16), jnp.float32),
            pltpu.VMEM((NBUF, SW * 128), jnp.int32),
            pltpu.VMEM((NBUF, SW * 128), jnp.int32),
            pltpu.VMEM((NBUF, SW * 128, 16), jnp.float32),
            pltpu.VMEM((2, UCH, 16), jnp.float32),
            pltpu.VMEM((2, UCH, 16), jnp.float32),
            pltpu.VMEM((2, UCH, 16), jnp.float32),
        ] + [pltpu.SemaphoreType.DMA] * (4 * NBUF + 4),
    )


def _final_body(w0_ref, w1_ref, dsq_ref, out_ref):
    z = jnp.concatenate([w0_ref[...], w1_ref[...]], axis=1) * dsq_ref[...]
    m = jnp.max(z, axis=1, keepdims=True)
    e = jnp.exp(z - m)
    s = jnp.sum(e, axis=1, keepdims=True)
    out_ref[...] = z - m - jnp.log(s)


def _final(w0, w1, dsq):
    return pl.pallas_call(
        _final_body,
        grid=(N // OUT_BLK,),
        in_specs=[
            pl.BlockSpec((OUT_BLK, 16), lambda i: (i, 0)),
            pl.BlockSpec((OUT_BLK, 16), lambda i: (i, 0)),
            pl.BlockSpec((OUT_BLK, 1), lambda i: (i, 0)),
        ],
        out_specs=pl.BlockSpec((OUT_BLK, C), lambda i: (i, 0)),
        out_shape=jax.ShapeDtypeStruct((N, C), jnp.float32),
    )(w0, w1, dsq)


def kernel(x, edge_index, W1, b1, W2, b2):
    pad = N + jnp.arange(E_PAD - E, dtype=edge_index.dtype) % (N_PAD - N)
    src2 = jnp.concatenate([edge_index[0], pad]).reshape(EROWS, 128)
    dst2 = jnp.concatenate([edge_index[1], pad]).reshape(EROWS, 128)
    xp = jnp.pad(x, ((0, N_PAD - N), (0, 0)))
    h = _mlp(xp, W1, b1, W2, b2)
    zeros_n = jnp.zeros((N_PAD,), jnp.float32)
    deg0 = _deg()(dst2, zeros_n)
    w0, w1, g0, g1, sb, dsq = _prep(h, deg0.reshape(N_PAD, 1))
    w_init = jnp.stack([w0, w1])
    gb = jnp.stack([g0, g1])
    w_fin = _prop()(src2.reshape(E_PAD), dst2.reshape(E_PAD), sb, gb,
                    w_init)
    return _final(w_fin[0], w_fin[1], dsq)


# speedup vs baseline: 1.0025x; 1.0025x over previous
import functools

import jax
import jax.numpy as jnp
from jax import lax
from jax.experimental import pallas as pl
from jax.experimental.pallas import tpu as pltpu
from jax.experimental.pallas import tpu_sc as plsc

N = 100000
E = 1600000
D = 128
H = 64
C = 32
K = 10
ALPHA = 0.1

NCORE = 2
NTILE = 16
NPT = 6256
N_PAD = NTILE * NPT
ROWS_PT = 792
EROWS = NTILE * ROWS_PT
E_PAD = EROWS * 128
SW = 3
NBUF = 3
NWIN = ROWS_PT // SW
NTRI = NWIN // NBUF
DSW = 8
NSW = ROWS_PT // DSW
UCH = 92
NUCH = NPT // UCH
NUPAIR = NUCH // 2

MLP_BLK = 4352
OUT_BLK = 2000

_MESH = dict(core_axis_name="c", subcore_axis_name="s", num_cores=NCORE,
             num_subcores=NTILE)


def _mlp_body(x_ref, w1_ref, b1_ref, w2_ref, b2_ref, out_ref):
    h = jnp.maximum(
        jnp.dot(x_ref[...], w1_ref[...], preferred_element_type=jnp.float32)
        + b1_ref[...], 0.0)
    out_ref[...] = (
        jnp.dot(h, w2_ref[...], preferred_element_type=jnp.float32)
        + b2_ref[...])


def _mlp(xp, W1, b1, W2, b2):
    return pl.pallas_call(
        _mlp_body,
        grid=(N_PAD // MLP_BLK,),
        in_specs=[
            pl.BlockSpec((MLP_BLK, D), lambda i: (i, 0)),
            pl.BlockSpec((D, H), lambda i: (0, 0)),
            pl.BlockSpec((1, H), lambda i: (0, 0)),
            pl.BlockSpec((H, C), lambda i: (0, 0)),
            pl.BlockSpec((1, C), lambda i: (0, 0)),
        ],
        out_specs=pl.BlockSpec((MLP_BLK, C), lambda i: (i, 0)),
        out_shape=jax.ShapeDtypeStruct((N_PAD, C), jnp.float32),
    )(xp, W1, b1[None, :], W2, b2[None, :])


def _deg_body(dst2, zeros_n, deg_out, dacc, didx, ones_v, zbuf):
    cid = lax.axis_index("c")
    sid = lax.axis_index("s")

    @pl.when(cid == 0)
    def _():
        node_lo = sid * NPT
        pltpu.sync_copy(zeros_n.at[pl.ds(node_lo, NPT)], zbuf)
        pltpu.sync_copy(zbuf, dacc.at[pl.ds(node_lo, NPT)])
        for u in range(8):
            ones_v[pl.ds(u * 16, 16)] = jnp.full((16,), 1.0, jnp.float32)
        plsc.subcore_barrier()
        row0 = sid * ROWS_PT

        def swin(i, carry):
            r = row0 + i * DSW
            pltpu.sync_copy(dst2.at[pl.ds(r, DSW)], didx)
            for j in range(DSW):
                pltpu.sync_copy(ones_v, dacc.at[didx.at[j]], add=True)
            return carry

        lax.fori_loop(0, NSW, swin, 0)
        plsc.subcore_barrier()
        pltpu.sync_copy(dacc.at[pl.ds(node_lo, NPT)], zbuf)
        pltpu.sync_copy(zbuf, deg_out.at[pl.ds(node_lo, NPT)])


@functools.cache
def _deg():
    return pl.kernel(
        _deg_body,
        out_type=jax.ShapeDtypeStruct((N_PAD,), jnp.float32),
        mesh=plsc.VectorSubcoreMesh(**_MESH),
        compiler_params=pltpu.CompilerParams(use_tc_tiling_on_sc=False),
        scratch_types=[
            pltpu.VMEM_SHARED((N_PAD,), jnp.float32),
            pltpu.VMEM((DSW, 128), jnp.int32),
            pltpu.VMEM((128,), jnp.float32),
            pltpu.VMEM((NPT,), jnp.float32),
        ],
    )


def _prep_body(h_ref, dg_ref, w0_ref, w1_ref, g0_ref, g1_ref, sb_ref,
               dsq_ref):
    deg = dg_ref[...] + 1.0
    dinv = jax.lax.rsqrt(deg)
    w = dinv * h_ref[...]
    w0_ref[...] = w[:, :16]
    w1_ref[...] = w[:, 16:]
    g0_ref[...] = ALPHA * w[:, :16]
    g1_ref[...] = ALPHA * w[:, 16:]
    sb_ref[...] = jnp.broadcast_to((1.0 - ALPHA) * dinv * dinv,
                                   (MLP_BLK, 16))
    dsq_ref[...] = deg * dinv


def _prep(h, dg):
    f32 = jnp.float32
    return pl.pallas_call(
        _prep_body,
        grid=(N_PAD // MLP_BLK,),
        in_specs=[
            pl.BlockSpec((MLP_BLK, C), lambda i: (i, 0)),
            pl.BlockSpec((MLP_BLK, 1), lambda i: (i, 0)),
        ],
        out_specs=[
            pl.BlockSpec((MLP_BLK, 16), lambda i: (i, 0)),
            pl.BlockSpec((MLP_BLK, 16), lambda i: (i, 0)),
            pl.BlockSpec((MLP_BLK, 16), lambda i: (i, 0)),
            pl.BlockSpec((MLP_BLK, 16), lambda i: (i, 0)),
            pl.BlockSpec((MLP_BLK, 16), lambda i: (i, 0)),
            pl.BlockSpec((MLP_BLK, 1), lambda i: (i, 0)),
        ],
        out_shape=[
            jax.ShapeDtypeStruct((N_PAD, 16), f32),
            jax.ShapeDtypeStruct((N_PAD, 16), f32),
            jax.ShapeDtypeStruct((N_PAD, 16), f32),
            jax.ShapeDtypeStruct((N_PAD, 16), f32),
            jax.ShapeDtypeStruct((N_PAD, 16), f32),
            jax.ShapeDtypeStruct((N_PAD, 1), f32),
        ],
    )(h, dg)


def _prop_body(src1, dst1, sb, gb, w_init, w_out,
               acc, sidx, didx, rows, abuf, sbuf, gbuf, *sems):
    cid = lax.axis_index("c")
    sid = lax.axis_index("s")
    node_lo = sid * NPT
    sisems = sems[:NBUF]
    disems = sems[NBUF:2 * NBUF]
    gsems = sems[2 * NBUF:3 * NBUF]
    ssems = sems[3 * NBUF:4 * NBUF]
    usems = sems[4 * NBUF:4 * NBUF + 2]
    osems = sems[4 * NBUF + 2:4 * NBUF + 4]

    def init(ci, c2):
        lo = node_lo + ci * UCH
        pltpu.sync_copy(w_init.at[cid].at[pl.ds(lo, UCH)], abuf.at[0])
        pltpu.sync_copy(abuf.at[0], acc.at[pl.ds(lo, UCH)])
        pltpu.sync_copy(abuf.at[0], w_out.at[cid].at[pl.ds(lo, UCH)])
        return c2

    lax.fori_loop(0, NUCH, init, 0)
    plsc.subcore_barrier()
    row0 = sid * ROWS_PT
    tbl = w_out.at[cid]

    WB = SW * 128

    def pre_sidx(w, b):
        e0 = (row0 + w * SW) * 128
        pltpu.async_copy(src1.at[pl.ds(e0, WB)], sidx.at[b], sisems[b])

    def wait_sidx(b):
        pltpu.make_async_copy(src1.at[pl.ds(0, WB)], sidx.at[b],
                              sisems[b]).wait()

    def drain_scatter(b):
        pltpu.make_async_copy(rows.at[b], acc.at[didx.at[b]],
                              ssems[b]).wait()

    def step(k, carry):
        for b in range(NBUF):
            pre_sidx(b, b)

        def tri(i, c2):
            gds = []
            dds = []
            for b in range(NBUF):
                wait_sidx(b)

                @pl.when(i > 0)
                def _(b=b):
                    drain_scatter(b)

                e0 = (row0 + (NBUF * i + b) * SW) * 128
                dds.append(pltpu.async_copy(dst1.at[pl.ds(e0, WB)],
                                            didx.at[b], disems[b]))
                gds.append(pltpu.async_copy(tbl.at[sidx.at[b]],
                                            rows.at[b], gsems[b]))
            for b in range(NBUF):
                gds[b].wait()
                dds[b].wait()
                pltpu.async_copy(rows.at[b], acc.at[didx.at[b]],
                                 ssems[b], add=True)

                @pl.when(i < NTRI - 1)
                def _(b=b):
                    pre_sidx(NBUF * i + b + NBUF, b)

            return c2

        lax.fori_loop(0, NTRI, tri, 0)
        for b in range(NBUF):
            drain_scatter(b)
        plsc.subcore_barrier()

        def issue_in(ci, b):
            lo = node_lo + ci * UCH
            return [
                pltpu.async_copy(sb.at[pl.ds(lo, UCH)], sbuf.at[b],
                                 usems[b]),
                pltpu.async_copy(gb.at[cid].at[pl.ds(lo, UCH)], gbuf.at[b],
                                 usems[b]),
            ]

        def compute(b):
            ab, sbf, gbf = abuf.at[b], sbuf.at[b], gbuf.at[b]

            def rowblk(jb, c3):
                for u in range(4):
                    i = jb * 4 + u
                    ab[i] = sbf[i] * ab[i] + gbf[i]
                return c3

            lax.fori_loop(0, UCH // 4, rowblk, 0)

        def issue_out(ci, b):
            lo = node_lo + ci * UCH
            d_ = pltpu.async_copy(abuf.at[b],
                                  w_out.at[cid].at[pl.ds(lo, UCH)],
                                  osems[b])
            pltpu.sync_copy(abuf.at[b], acc.at[pl.ds(lo, UCH)])
            return [d_]

        def upair(i, c2):
            din0 = issue_in(2 * i, 0)
            din1 = issue_in(2 * i + 1, 1)
            pltpu.sync_copy(acc.at[pl.ds(node_lo + 2 * i * UCH, UCH)],
                            abuf.at[0])
            pltpu.sync_copy(acc.at[pl.ds(node_lo + (2 * i + 1) * UCH, UCH)],
                            abuf.at[1])
            for d_ in din0:
                d_.wait()
            compute(0)
            dout0 = issue_out(2 * i, 0)
            for d_ in din1:
                d_.wait()
            compute(1)
            dout1 = issue_out(2 * i + 1, 1)
            for d_ in dout0:
                d_.wait()
            for d_ in dout1:
                d_.wait()
            return c2

        lax.fori_loop(0, NUPAIR, upair, 0)
        plsc.subcore_barrier()
        return carry

    lax.fori_loop(0, K, step, 0)


@functools.cache
def _prop():
    return pl.kernel(
        _prop_body,
        out_type=jax.ShapeDtypeStruct((NCORE, N_PAD, 16), jnp.float32),
        mesh=plsc.VectorSubcoreMesh(**_MESH),
        compiler_params=pltpu.CompilerParams(use_tc_tiling_on_sc=False),
        scratch_types=[
            pltpu.VMEM_SHARED((N_PAD, 16), jnp.float32),
            pltpu.VMEM((NBUF, SW * 128), jnp.int32),
            pltpu.VMEM((NBUF, SW * 128), jnp.int32),
            pltpu.VMEM((NBUF, SW * 128, 16), jnp.float32),
            pltpu.VMEM((2, UCH, 16), jnp.float32),
            pltpu.VMEM((2, UCH, 16), jnp.float32),
            pltpu.VMEM((2, UCH, 16), jnp.float32),
        ] + [pltpu.SemaphoreType.DMA] * (4 * NBUF + 4),
    )


def _final_body(w0_ref, w1_ref, dsq_ref, out_ref):
    z = jnp.concatenate([w0_ref[...], w1_ref[...]], axis=1) * dsq_ref[...]
    m = jnp.max(z, axis=1, keepdims=True)
    e = jnp.exp(z - m)
    s = jnp.sum(e, axis=1, keepdims=True)
    out_ref[...] = z - m - jnp.log(s)


def _final(w0, w1, dsq):
    return pl.pallas_call(
        _final_body,
        grid=(N // OUT_BLK,),
        in_specs=[
            pl.BlockSpec((OUT_BLK, 16), lambda i: (i, 0)),
            pl.BlockSpec((OUT_BLK, 16), lambda i: (i, 0)),
            pl.BlockSpec((OUT_BLK, 1), lambda i: (i, 0)),
        ],
        out_specs=pl.BlockSpec((OUT_BLK, C), lambda i: (i, 0)),
        out_shape=jax.ShapeDtypeStruct((N, C), jnp.float32),
    )(w0, w1, dsq)


def kernel(x, edge_index, W1, b1, W2, b2):
    pad = N + jnp.arange(E_PAD - E, dtype=edge_index.dtype) % (N_PAD - N)
    src2 = jnp.concatenate([edge_index[0], pad]).reshape(EROWS, 128)
    dst2 = jnp.concatenate([edge_index[1], pad]).reshape(EROWS, 128)
    xp = jnp.pad(x, ((0, N_PAD - N), (0, 0)))
    h = _mlp(xp, W1, b1, W2, b2)
    zeros_n = jnp.zeros((N_PAD,), jnp.float32)
    deg0 = _deg()(dst2, zeros_n)
    w0, w1, g0, g1, sb, dsq = _prep(h, deg0.reshape(N_PAD, 1))
    w_init = jnp.stack([w0, w1])
    gb = jnp.stack([g0, g1])
    w_fin = _prop()(src2.reshape(E_PAD), dst2.reshape(E_PAD), sb, gb,
                    w_init)
    return _final(w_fin[0], w_fin[1], dsq)
